# initial kernel scaffold (unmeasured)
import jax
import jax.numpy as jnp
from jax import lax
from jax.experimental import pallas as pl
from jax.experimental.pallas import tpu as pltpu


def kernel(
    x,
):
    def body(*refs):
        pass

    out_shape = jax.ShapeDtypeStruct(..., jnp.float32)
    return pl.pallas_call(body, out_shape=out_shape)(...)


# baseline (device time: 61941 ns/iter reference)
import jax
import jax.numpy as jnp
from jax import lax
from jax.experimental import pallas as pl
from jax.experimental.pallas import tpu as pltpu

N_DEV = 4
K = 16
NEG = float("-inf")


def _topk_desc(work, k):
    rows, cols = work.shape
    iota = lax.broadcasted_iota(jnp.int32, (rows, cols), 1)
    out_cols = []
    for j in range(k):
        m = jnp.max(work, axis=1, keepdims=True)
        out_cols.append(m)
        if j < k - 1:
            idx = jnp.where(work == m, iota, cols)
            first = jnp.min(idx, axis=1, keepdims=True)
            work = jnp.where(idx == first, NEG, work)
    return jnp.concatenate(out_cols, axis=1)


def kernel(x):
    m, n = x.shape

    def body(x_ref, out_ref, comm_ref, send_sems, recv_sems):
        my_pos = lax.axis_index("i")
        left = lax.rem(my_pos + N_DEV - 1, N_DEV)
        right = lax.rem(my_pos + 1, N_DEV)

        barrier_sem = pltpu.get_barrier_semaphore()
        for nbr in (left, right):
            pl.semaphore_signal(
                barrier_sem, inc=1,
                device_id=(nbr,), device_id_type=pl.DeviceIdType.MESH,
            )
        pl.semaphore_wait(barrier_sem, 2)

        comm_ref[0, :, :] = _topk_desc(x_ref[:, :], K)

        for h in range(N_DEV - 1):
            rdma = pltpu.make_async_remote_copy(
                src_ref=comm_ref.at[h],
                dst_ref=comm_ref.at[h + 1],
                send_sem=send_sems.at[h],
                recv_sem=recv_sems.at[h],
                device_id=(right,),
                device_id_type=pl.DeviceIdType.MESH,
            )
            rdma.start()
            rdma.wait()

        cand = jnp.concatenate([comm_ref[i] for i in range(N_DEV)], axis=1)
        out_ref[:, :] = _topk_desc(cand, K)

    return pl.pallas_call(
        body,
        out_shape=jax.ShapeDtypeStruct((m, K), jnp.float32),
        in_specs=[pl.BlockSpec(memory_space=pltpu.VMEM)],
        out_specs=pl.BlockSpec(memory_space=pltpu.VMEM),
        scratch_shapes=[
            pltpu.VMEM((N_DEV, m, K), jnp.float32),
            pltpu.SemaphoreType.DMA((N_DEV - 1,)),
            pltpu.SemaphoreType.DMA((N_DEV - 1,)),
        ],
        compiler_params=pltpu.CompilerParams(collective_id=0),
    )(x)


# device time: 32602 ns/iter; 1.8999x vs baseline; 1.8999x over previous
import jax
import jax.numpy as jnp
from jax import lax
from jax.experimental import pallas as pl
from jax.experimental.pallas import tpu as pltpu

N_DEV = 4
K = 16
CHUNK = 128
NEG = float("-inf")


def _bitonic_sort_desc(L):
    L = list(L)
    k = len(L)
    d = k // 2
    while d >= 1:
        for blk in range(0, k, 2 * d):
            for i in range(blk, blk + d):
                hi = jnp.maximum(L[i], L[i + d])
                lo = jnp.minimum(L[i], L[i + d])
                L[i], L[i + d] = hi, lo
        d //= 2
    return L


def _merge_desc(A, B):
    return _bitonic_sort_desc(A + B[::-1])


def _merge_topk_desc(A, B):
    k = len(A)
    top = [jnp.maximum(A[i], B[k - 1 - i]) for i in range(k)]
    return _bitonic_sort_desc(top)


def _lroll(a, s):
    return jnp.concatenate([a[:, s:], a[:, :s]], axis=1)


def _topk_extract_desc(work, k):
    rows, cols = work.shape
    iota = lax.broadcasted_iota(jnp.int32, (rows, cols), 1)
    out_cols = []
    for j in range(k):
        m = jnp.max(work, axis=1, keepdims=True)
        out_cols.append(m)
        if j < k - 1:
            idx = jnp.where(work == m, iota, cols)
            first = jnp.min(idx, axis=1, keepdims=True)
            work = jnp.where(idx == first, NEG, work)
    return jnp.concatenate(out_cols, axis=1)


def _local_topk(x):
    rows, n = x.shape
    chunks = [x[:, c * CHUNK:(c + 1) * CHUNK] for c in range(n // CHUNK)]

    lists = [[c] for c in chunks]
    while len(lists) > 1:
        nxt = []
        for j in range(0, len(lists), 2):
            A, B = lists[j], lists[j + 1]
            if len(A) >= K:
                nxt.append(_merge_topk_desc(A, B))
            else:
                nxt.append(_merge_desc(A, B))
        lists = nxt
    R = lists[0]

    w = CHUNK
    while w > K:
        h = w // 2
        top = [jnp.maximum(R[i], _lroll(R[K - 1 - i], h)) for i in range(K)]
        R = _bitonic_sort_desc(top)
        w = h

    cand = jnp.concatenate([r[:, :K] for r in R], axis=1)
    return _topk_extract_desc(cand, K)


def kernel(x):
    m, n = x.shape

    def body(x_ref, out_ref, cand_ref, send_sems, recv_sems):
        my_pos = lax.axis_index("i")

        barrier_sem = pltpu.get_barrier_semaphore()
        for o in range(1, N_DEV):
            pl.semaphore_signal(
                barrier_sem, inc=1,
                device_id=(lax.rem(my_pos + o, N_DEV),),
                device_id_type=pl.DeviceIdType.MESH,
            )

        local = _local_topk(x_ref[:, :])
        cand_ref[pl.ds(my_pos, 1)] = local[None, :, :]

        pl.semaphore_wait(barrier_sem, N_DEV - 1)

        rdmas = []
        for o in range(1, N_DEV):
            peer = lax.rem(my_pos + o, N_DEV)
            rdma = pltpu.make_async_remote_copy(
                src_ref=cand_ref.at[my_pos],
                dst_ref=cand_ref.at[my_pos],
                send_sem=send_sems.at[o - 1],
                recv_sem=recv_sems.at[o - 1],
                device_id=(peer,),
                device_id_type=pl.DeviceIdType.MESH,
            )
            rdma.start()
            rdmas.append(rdma)
        for rdma in rdmas:
            rdma.wait()

        gathered = jnp.concatenate(
            [cand_ref[i] for i in range(N_DEV)], axis=1
        )
        out_ref[:, :] = _topk_extract_desc(gathered, K)

    return pl.pallas_call(
        body,
        out_shape=jax.ShapeDtypeStruct((m, K), jnp.float32),
        in_specs=[pl.BlockSpec(memory_space=pltpu.VMEM)],
        out_specs=pl.BlockSpec(memory_space=pltpu.VMEM),
        scratch_shapes=[
            pltpu.VMEM((N_DEV, m, K), jnp.float32),
            pltpu.SemaphoreType.DMA((N_DEV - 1,)),
            pltpu.SemaphoreType.DMA((N_DEV - 1,)),
        ],
        compiler_params=pltpu.CompilerParams(collective_id=0),
    )(x)


# device time: 25440 ns/iter; 2.4348x vs baseline; 1.2815x over previous
import jax
import jax.numpy as jnp
from jax import lax
from jax.experimental import pallas as pl
from jax.experimental.pallas import tpu as pltpu

N_DEV = 4
K = 16
CHUNK = 128
NEG = float("-inf")


def _bitonic_sort_desc(L):
    L = list(L)
    k = len(L)
    d = k // 2
    while d >= 1:
        for blk in range(0, k, 2 * d):
            for i in range(blk, blk + d):
                hi = jnp.maximum(L[i], L[i + d])
                lo = jnp.minimum(L[i], L[i + d])
                L[i], L[i + d] = hi, lo
        d //= 2
    return L


def _merge_desc(A, B):
    return _bitonic_sort_desc(A + B[::-1])


def _merge_topk_desc(A, B):
    k = len(A)
    top = [jnp.maximum(A[i], B[k - 1 - i]) for i in range(k)]
    return _bitonic_sort_desc(top)


def _lroll(a, s):
    return jnp.concatenate([a[:, s:], a[:, :s]], axis=1)


def _topk_extract_desc(work, k):
    out_cols = []
    for j in range(k):
        m = jnp.max(work, axis=1, keepdims=True)
        out_cols.append(m)
        if j < k - 1:
            work = jnp.where(work == m, NEG, work)
    return jnp.concatenate(out_cols, axis=1)


def _local_topk(x):
    rows, n = x.shape
    chunks = [x[:, c * CHUNK:(c + 1) * CHUNK] for c in range(n // CHUNK)]

    lists = [[c] for c in chunks]
    while len(lists) > 1:
        nxt = []
        for j in range(0, len(lists), 2):
            A, B = lists[j], lists[j + 1]
            if len(A) >= K:
                nxt.append(_merge_topk_desc(A, B))
            else:
                nxt.append(_merge_desc(A, B))
        lists = nxt
    R = lists[0]

    w = CHUNK
    while w > K:
        h = w // 2
        top = [jnp.maximum(R[i], _lroll(R[K - 1 - i], h)) for i in range(K)]
        R = top if h == K else _bitonic_sort_desc(top)
        w = h

    cand = jnp.concatenate([r[:, :K] for r in R], axis=1)
    return _topk_extract_desc(cand, K)


def kernel(x):
    m, n = x.shape

    def body(x_ref, out_ref, cand_ref, send_sems, recv_sems):
        my_pos = lax.axis_index("i")

        barrier_sem = pltpu.get_barrier_semaphore()
        for o in range(1, N_DEV):
            pl.semaphore_signal(
                barrier_sem, inc=1,
                device_id=(lax.rem(my_pos + o, N_DEV),),
                device_id_type=pl.DeviceIdType.MESH,
            )

        local = _local_topk(x_ref[:, :])
        cand_ref[pl.ds(my_pos, 1)] = local[None, :, :]

        pl.semaphore_wait(barrier_sem, N_DEV - 1)

        rdmas = []
        for o in range(1, N_DEV):
            peer = lax.rem(my_pos + o, N_DEV)
            rdma = pltpu.make_async_remote_copy(
                src_ref=cand_ref.at[my_pos],
                dst_ref=cand_ref.at[my_pos],
                send_sem=send_sems.at[o - 1],
                recv_sem=recv_sems.at[o - 1],
                device_id=(peer,),
                device_id_type=pl.DeviceIdType.MESH,
            )
            rdma.start()
            rdmas.append(rdma)
        for rdma in rdmas:
            rdma.wait()

        gathered = jnp.concatenate(
            [cand_ref[i] for i in range(N_DEV)], axis=1
        )
        out_ref[:, :] = _topk_extract_desc(gathered, K)

    return pl.pallas_call(
        body,
        out_shape=jax.ShapeDtypeStruct((m, K), jnp.float32),
        in_specs=[pl.BlockSpec(memory_space=pltpu.VMEM)],
        out_specs=pl.BlockSpec(memory_space=pltpu.VMEM),
        scratch_shapes=[
            pltpu.VMEM((N_DEV, m, K), jnp.float32),
            pltpu.SemaphoreType.DMA((N_DEV - 1,)),
            pltpu.SemaphoreType.DMA((N_DEV - 1,)),
        ],
        compiler_params=pltpu.CompilerParams(collective_id=0),
    )(x)
